# Initial kernel scaffold; baseline (speedup 1.0000x reference)
import jax
import jax.numpy as jnp
from jax.experimental import pallas as pl


def kernel(zL, params):
    raise NotImplementedError("write your pallas kernel here")


# speedup vs baseline: 5.1696x; 5.1696x over previous
import functools
import jax
import jax.numpy as jnp
from jax import lax
from jax.experimental import pallas as pl

B = 1
DEPTH = 16
L = 512
D_MODEL = 768
N_HEADS = 12
HEAD_DIM = D_MODEL // N_HEADS
N_FEATURES = 4096
TOPK = 64
EPS = 1e-5
INV_H = float(1.0 / 768.0)
BITREV16 = [0, 8, 4, 12, 2, 10, 6, 14, 1, 9, 5, 13, 3, 11, 7, 15]


def _bdot(a, b):
    return jnp.dot(a.astype(jnp.bfloat16), b.astype(jnp.bfloat16),
                   preferred_element_type=jnp.float32)


def _bdot_k256(a, b):
    out = None
    for st in range(0, a.shape[-1], 256):
        p = jnp.dot(a[:, st:st + 256].astype(jnp.bfloat16),
                    b[st:st + 256, :].astype(jnp.bfloat16),
                    preferred_element_type=jnp.float32)
        out = p if out is None else out + p
    return out


def _softmax_den_512(e):
    n = e.shape[-1]
    acc = e[:, 0:128]
    for t in range(1, n // 128):
        acc = acc + e[:, 128 * t:128 * (t + 1)]
    a = acc
    for sh in (1, 2, 4):
        a = a + jnp.concatenate([a[:, sh:], a[:, :sh]], axis=1)
    den = a[:, 0:1]
    for k in range(1, 16):
        den = den + a[:, 8 * k:8 * k + 1]
    return den


def _var_sum(x):
    return _softmax_den_512(x)


def _bdot_nt(a, b):
    return lax.dot_general(a.astype(jnp.bfloat16), b.astype(jnp.bfloat16),
                           (((1,), (1,)), ((), ())),
                           preferred_element_type=jnp.float32)


def _tree_sum(x):
    n = x.shape[-1]
    acc = x[:, 0:128]
    for t in range(1, n // 128):
        acc = acc + x[:, 128 * t:128 * (t + 1)]
    c = acc[:, 0:8]
    for k in range(1, 16):
        c = c + acc[:, 8 * k:8 * k + 8]
    c = c[:, 0:4] + c[:, 4:8]
    c = c[:, 0:2] + c[:, 2:4]
    return c[:, 0:1] + c[:, 1:2]


def _layernorm(x, w, b):
    mu = _tree_sum(x) * INV_H
    d = x - mu
    var = _var_sum(d * d) * INV_H
    return d / jnp.sqrt(var + EPS) * w + b


def _spatial_attn_body(x_ref, xq_ref, wq_ref, wk_ref, wv_ref, wo_ref,
                       lw_ref, lb_ref, o_ref):
    x = x_ref[0]
    xq = xq_ref[0]
    ln = _layernorm(x, lw_ref[0], lb_ref[0])
    lnq = _layernorm(xq, lw_ref[0], lb_ref[0])
    q = _bdot_k256(lnq, wq_ref[...]).astype(jnp.bfloat16)
    k = _bdot_k256(ln, wk_ref[...]).astype(jnp.bfloat16)
    v = _bdot_k256(ln, wv_ref[...]).astype(jnp.bfloat16)
    ohs = []
    for h in range(N_HEADS):
        sl = slice(h * HEAD_DIM, (h + 1) * HEAD_DIM)
        s = _bdot_nt(q[:, sl], k[:, sl]) * 0.125
        m = jnp.max(s, axis=-1, keepdims=True)
        e = jnp.exp(s - m)
        w = e / _softmax_den_512(e)
        ohs.append(_bdot(w, v[:, sl]).astype(jnp.bfloat16))
    o = jnp.concatenate(ohs, axis=-1)
    o_ref[:, 0, 0, :] = xq + _bdot_k256(o, wo_ref[...])


def _depth_attn_body(x_ref, wq_ref, wk_ref, wv_ref, wo_ref, lw_ref, lb_ref,
                     o_ref, *, lchunk):
    n = lchunk * DEPTH
    x = x_ref[...].reshape(n, D_MODEL)
    ln = _layernorm(x, lw_ref[0], lb_ref[0])
    q = _bdot_k256(ln, wq_ref[...]).astype(jnp.bfloat16)
    k = _bdot_k256(ln, wk_ref[...]).astype(jnp.bfloat16)
    v = _bdot_k256(ln, wv_ref[...]).astype(jnp.bfloat16)
    rr = lax.broadcasted_iota(jnp.int32, (n, n), 0)
    cc = lax.broadcasted_iota(jnp.int32, (n, n), 1)
    same = (rr // DEPTH) == (cc // DEPTH)
    keep = same & ((rr % DEPTH) >= (cc % DEPTH))
    ninf = jnp.float32(-jnp.inf)
    ohs = []
    for h in range(N_HEADS):
        sl = slice(h * HEAD_DIM, (h + 1) * HEAD_DIM)
        kh = k[:, sl]
        s = _bdot_nt(q[:, sl], kh) * 0.125
        s = jnp.where(keep, s, ninf)
        m = jnp.max(s, axis=-1, keepdims=True)
        e = jnp.exp(s - m)
        w = e / _tree_sum(e)
        ohs.append(_bdot(w, v[:, sl]).astype(jnp.bfloat16))
    o = jnp.concatenate(ohs, axis=-1)
    o_ref[...] = (x + _bdot_k256(o, wo_ref[...])).reshape(lchunk, DEPTH, D_MODEL)


def _dict_topk_body(zl_ref, x_ref, wd_ref, bp_ref, be_ref, o_ref):
    xs = zl_ref[0] - x_ref[...] - bp_ref[0]
    logits = _bdot_k256(xs, wd_ref[...]) + be_ref[0]
    z = jnp.maximum(logits, 0.0)
    zi = lax.bitcast_convert_type(z, jnp.int32)
    prefix = jnp.zeros((z.shape[0], 1), dtype=jnp.int32)
    for bit in range(30, -1, -1):
        cand = prefix | (1 << bit)
        cnt = jnp.sum((zi >= cand).astype(jnp.int32), axis=1, keepdims=True)
        prefix = jnp.where(cnt >= TOPK, cand, prefix)
    o_ref[0] = jnp.where(zi >= prefix, z, 0.0)


def kernel(zL, params):
    p = params
    Bz, D, Lz, H = zL.shape
    qtok = jnp.broadcast_to(p['query_token'][None, None, None, :],
                            (Bz, 1, Lz, H))
    x0 = jnp.concatenate([qtok, zL[:, :-1, :, :]], axis=1)
    x0 = x0.reshape(D, Lz, H)

    lw = p['norm_l_w'].reshape(1, H)
    lb = p['norm_l_b'].reshape(1, H)
    dw = p['norm_d_w'].reshape(1, H)
    db = p['norm_d_b'].reshape(1, H)
    w_full = pl.BlockSpec((H, H), lambda *_: (0, 0))
    vec_h = pl.BlockSpec((1, H), lambda *_: (0, 0))

    QC = 128
    x1 = pl.pallas_call(
        _spatial_attn_body,
        grid=(D, Lz // QC),
        in_specs=[
            pl.BlockSpec((1, Lz, H), lambda d, t: (d, 0, 0)),
            pl.BlockSpec((1, QC, H), lambda d, t: (d, t, 0)),
            w_full, w_full, w_full, w_full, vec_h, vec_h,
        ],
        out_specs=pl.BlockSpec((QC, 1, 1, H), lambda d, t: (t, d, 0, 0)),
        out_shape=jax.ShapeDtypeStruct((Lz, D, 1, H), jnp.float32),
    )(x0, x0, *[w.T.astype(jnp.bfloat16) for w in
                (p['l_q'], p['l_k'], p['l_v'], p['l_o'])], lw, lb)
    x1 = x1.reshape(Lz, D, H)

    LCHUNK = 32
    x2 = pl.pallas_call(
        functools.partial(_depth_attn_body, lchunk=LCHUNK),
        grid=(Lz // LCHUNK,),
        in_specs=[
            pl.BlockSpec((LCHUNK, D, H), lambda t: (t, 0, 0)),
            w_full, w_full, w_full, w_full, vec_h, vec_h,
        ],
        out_specs=pl.BlockSpec((LCHUNK, D, H), lambda t: (t, 0, 0)),
        out_shape=jax.ShapeDtypeStruct((Lz, D, H), jnp.float32),
    )(x1, *[w.T.astype(jnp.bfloat16) for w in
            (p['d_q'], p['d_k'], p['d_v'], p['d_o'])], dw, db)

    ROWS = 256
    F = N_FEATURES
    out = pl.pallas_call(
        _dict_topk_body,
        grid=(D, Lz // ROWS),
        in_specs=[
            pl.BlockSpec((1, ROWS, H), lambda d, t: (d, t, 0)),
            pl.BlockSpec((ROWS, H), lambda d, t: (t, d)),
            pl.BlockSpec((H, F), lambda *_: (0, 0)),
            vec_h,
            pl.BlockSpec((1, F), lambda *_: (0, 0)),
        ],
        out_specs=pl.BlockSpec((1, ROWS, F), lambda d, t: (d, t, 0)),
        out_shape=jax.ShapeDtypeStruct((D, Lz, F), jnp.float32),
    )(zL.reshape(D, Lz, H), x2.reshape(Lz, D * H),
      p['dict_enc'].T.astype(jnp.bfloat16),
      p['bias_pre'].reshape(1, H), p['bias_enc'].reshape(1, F))

    return out.reshape(Bz, D, Lz, F)
